# baseline (speedup 1.0000x reference)
import functools

import jax
import jax.numpy as jnp
from jax import lax
from jax.experimental import pallas as pl
from jax.experimental.pallas import tpu as pltpu
from jax.experimental.pallas import tpu_sc as plsc

N = 10000
E = 320000
F = 128
H = 64
D = 128
NC = 2
NS = 16
L = 16
NP = 10240
RPT = NP // NS
ACCR = 648
QC = 800
QN = E // NC // QC
GQ = QC // L
EC = 64
LSZ = 12288 + 2 * EC
WC = RPT // EC


def _proj_body(x_ref, w_ref, al_ref, ar_ref, feat_ref, er_ref, m_ref):
    feat = jnp.dot(x_ref[...], w_ref[...], preferred_element_type=jnp.float32)
    feat_ref[...] = feat
    el = jnp.dot(feat, al_ref[...], preferred_element_type=jnp.float32)
    er = jnp.dot(feat, ar_ref[...], preferred_element_type=jnp.float32)
    er_ref[...] = er
    msum = jnp.max(el) + jnp.max(er)
    m = jnp.where(msum > 0, msum, 0.2 * msum)
    m_ref[...] = jnp.full((1, L), m, jnp.float32)


_proj = pl.pallas_call(
    _proj_body,
    out_shape=(
        jax.ShapeDtypeStruct((N, D), jnp.float32),
        jax.ShapeDtypeStruct((N, 1), jnp.float32),
        jax.ShapeDtypeStruct((1, L), jnp.float32),
    ),
)


def _sc_body(src_hbm, dst_hbm, er_hbm, m_hbm, feat_hbm, out_hbm,
             srcq0, dstq0, srcq1, dstq1, er_b, m_b, list_b,
             sidx0, sidx1, dstl0, dstl1, ab0, ab1, row0, row1, acc,
             qs0, qs1, gs0, gs1):
    c = lax.axis_index("c")
    s = lax.axis_index("s")
    lo = s * RPT

    pltpu.sync_copy(er_hbm.at[pl.ds(lo, RPT)], er_b.at[pl.ds(0, RPT)])
    pltpu.sync_copy(m_hbm, m_b)
    zv = jnp.zeros((L,), jnp.float32)
    er_b[pl.ds(RPT, L)] = zv
    m_sh = m_b[...]

    def zrow(r, _):
        for kk in range(D // L):
            acc[r, pl.ds(kk * L, L)] = zv
        return 0
    lax.fori_loop(0, ACCR, zrow, 0)

    iota = lax.iota(jnp.int32, L)
    colh = jnp.full((L,), H, jnp.int32)

    qsrc_ = (srcq0, srcq1)
    qdst_ = (dstq0, dstq1)
    qsem_ = (qs0, qs1)

    def stage_q(q, qb):
        pltpu.async_copy(src_hbm.at[c, q], qsrc_[qb], qsem_[qb])
        pltpu.async_copy(dst_hbm.at[c, q], qdst_[qb], qsem_[qb])

    def wait_q(qb):
        pltpu.make_async_copy(src_hbm.at[c, 0], qsrc_[qb], qsem_[qb]).wait()
        pltpu.make_async_copy(dst_hbm.at[c, 0], qdst_[qb], qsem_[qb]).wait()

    stage_q(0, 0)
    stage_q(1, 1)

    def scan_pair(q2, ptr):
        for qb in range(2):
            q = q2 * 2 + qb
            wait_q(qb)

            def group(g, p):
                srcv = qsrc_[qb][pl.ds(g * L, L)]
                dstv = qdst_[qb][pl.ds(g * L, L)]
                dl = dstv - lo
                msk = (dl >= 0) & (dl < RPT)
                packed = jnp.left_shift(srcv, 10) | jnp.where(msk, dl, 0)
                plsc.store_compressed(list_b.at[pl.ds(p, L)], packed, mask=msk)
                return p + plsc.all_reduce_population_count(msk)[0]

            ptr = lax.fori_loop(0, GQ, group, ptr)

            @pl.when(q + 2 < QN)
            def _():
                stage_q(q + 2, qb)
        return ptr

    ptr = lax.fori_loop(0, QN // 2, scan_pair, jnp.int32(0))

    sent = jnp.full((L,), RPT, jnp.int32)
    for w in range(2 * EC // L):
        list_b[pl.ds(ptr + w * L, L)] = sent
    nch2 = (ptr + 2 * EC - 1) // (2 * EC)

    sidx_ = (sidx0, sidx1)
    dstl_ = (dstl0, dstl1)
    ab_ = (ab0, ab1)
    row_ = (row0, row1)
    gsem_ = (gs0, gs1)

    def build(ch, b):
        for k in range(EC // L):
            v = list_b[pl.ds(ch * EC + k * L, L)]
            sidx_[b][pl.ds(k * L, L)] = jnp.right_shift(v, 10)
            dstl_[b][pl.ds(k * L, L)] = v & 1023
        pltpu.async_copy(feat_hbm.at[sidx_[b]], row_[b], gsem_[b])

    @pl.when(nch2 > 0)
    def _():
        build(0, 0)

    def chunk_pair(ch2, _):
        for b in range(2):
            ch = ch2 * 2 + b
            rb, dlb, ab = row_[b], dstl_[b], ab_[b]
            pltpu.make_async_copy(feat_hbm.at[sidx_[b]], rb, gsem_[b]).wait()

            @pl.when(ch + 1 < nch2 * 2)
            def _():
                build(ch + 1, 1 - b)

            for k in range(EC // L):
                elv = plsc.load_gather(rb, [iota + k * L, colh])
                erv = plsc.load_gather(er_b, [dlb[pl.ds(k * L, L)]])
                x = elv + erv
                e = jnp.where(x >= 0, x, 0.2 * x)
                a = jnp.exp(e - m_sh)
                ab[pl.ds(k * L, L)] = a

            def edge(t, _):
                a_s = ab[pl.ds(t, L)][0]
                dl_s = dlb[pl.ds(t, L)][0]
                for cc in range(H // L):
                    plsc.addupdate(acc.at[dl_s, pl.ds(cc * L, L)],
                                   rb[t, pl.ds(cc * L, L)] * a_s)
                av = jnp.where(iota == 0, a_s, 0.0)
                plsc.addupdate(acc.at[dl_s, pl.ds(H, L)], av)
                return 0
            lax.fori_loop(0, EC, edge, 0)
        return 0

    lax.fori_loop(0, nch2, chunk_pair, 0)

    for z in range(WC):
        pltpu.sync_copy(acc.at[pl.ds(z * EC, EC)],
                        out_hbm.at[c, pl.ds(lo + z * EC, EC)])


_sc_gat = functools.partial(
    pl.kernel,
    out_type=jax.ShapeDtypeStruct((NC, NP, D), jnp.float32),
    mesh=plsc.VectorSubcoreMesh(core_axis_name="c", subcore_axis_name="s"),
    compiler_params=pltpu.CompilerParams(needs_layout_passes=False),
    scratch_types=[
        pltpu.VMEM((QC,), jnp.int32),
        pltpu.VMEM((QC,), jnp.int32),
        pltpu.VMEM((QC,), jnp.int32),
        pltpu.VMEM((QC,), jnp.int32),
        pltpu.VMEM((RPT + L,), jnp.float32),
        pltpu.VMEM((L,), jnp.float32),
        pltpu.VMEM((LSZ,), jnp.int32),
        pltpu.VMEM((EC,), jnp.int32),
        pltpu.VMEM((EC,), jnp.int32),
        pltpu.VMEM((EC + L,), jnp.int32),
        pltpu.VMEM((EC + L,), jnp.int32),
        pltpu.VMEM((EC + L,), jnp.float32),
        pltpu.VMEM((EC + L,), jnp.float32),
        pltpu.VMEM((EC, D), jnp.float32),
        pltpu.VMEM((EC, D), jnp.float32),
        pltpu.VMEM((ACCR, D), jnp.float32),
        pltpu.SemaphoreType.DMA,
        pltpu.SemaphoreType.DMA,
        pltpu.SemaphoreType.DMA,
        pltpu.SemaphoreType.DMA,
    ],
)(_sc_body)


def _head_body(s_ref, bias_ref, fcw_ref, fcb_ref, y_ref):
    num = s_ref[0, :N, :H] + s_ref[1, :N, :H]
    den = s_ref[0, :N, H:H + 1] + s_ref[1, :N, H:H + 1]
    rst = num / (den + 1e-16)
    h = rst + bias_ref[...]
    h = jnp.where(h > 0, h, jnp.exp(jnp.minimum(h, 0.0)) - 1.0)
    hg = jnp.mean(h, axis=0, keepdims=True)
    logit = jnp.sum(hg * fcw_ref[...], axis=1, keepdims=True) + fcb_ref[...]
    y_ref[...] = 1.0 / (1.0 + jnp.exp(-logit))


_head = pl.pallas_call(
    _head_body,
    out_shape=jax.ShapeDtypeStruct((1, 1), jnp.float32),
)


def kernel(features, edge_index, W, attn_l, attn_r, bias, fc_W, fc_b):
    al = attn_l.reshape(H)
    ar = attn_r.reshape(H)
    Wp = jnp.concatenate(
        [W, (W @ al)[:, None], jnp.zeros((F, D - H - 1), jnp.float32)], axis=1)
    alp = jnp.pad(al[:, None], ((0, D - H), (0, 0)))
    arp = jnp.pad(ar[:, None], ((0, D - H), (0, 0)))
    feat, er, m = _proj(features, Wp, alp, arp)
    er_p = jnp.pad(er.reshape(N), (0, NP - N))
    partials = _sc_gat(edge_index[0].reshape(NC, QN, QC),
                       edge_index[1].reshape(NC, QN, QC),
                       er_p, m.reshape(L), feat)
    return _head(partials, bias.reshape(1, H), fc_W, fc_b.reshape(1, 1))


# speedup vs baseline: 1.3645x; 1.3645x over previous
import functools

import jax
import jax.numpy as jnp
from jax import lax
from jax.experimental import pallas as pl
from jax.experimental.pallas import tpu as pltpu
from jax.experimental.pallas import tpu_sc as plsc

N = 10000
E = 320000
F = 128
H = 64
D = 128
NC = 2
NS = 16
L = 16
NP = 10240
RPT = NP // NS
ACCR = 648
QC = 800
QN = E // NC // QC
GQ = QC // L
EC = 64
LSZ = 12288 + 2 * EC
WC = RPT // EC


def _proj_body(x_ref, w_ref, al_ref, ar_ref, feat_ref, er_ref, m_ref):
    feat = jnp.dot(x_ref[...], w_ref[...], preferred_element_type=jnp.float32)
    feat_ref[...] = feat
    el = jnp.dot(feat, al_ref[...], preferred_element_type=jnp.float32)
    er = jnp.dot(feat, ar_ref[...], preferred_element_type=jnp.float32)
    er_ref[...] = er
    msum = jnp.max(el) + jnp.max(er)
    m = jnp.where(msum > 0, msum, 0.2 * msum)
    m_ref[...] = jnp.full((1, L), m, jnp.float32)


_proj = pl.pallas_call(
    _proj_body,
    out_shape=(
        jax.ShapeDtypeStruct((N, D), jnp.float32),
        jax.ShapeDtypeStruct((N, 1), jnp.float32),
        jax.ShapeDtypeStruct((1, L), jnp.float32),
    ),
)


def _sc_body(src_hbm, dst_hbm, er_hbm, m_hbm, feat_hbm, out_hbm,
             srcq0, dstq0, srcq1, dstq1, er_b, m_b, list_b,
             sidx0, sidx1, dstl0, dstl1, ab0, ab1, row0, row1, acc,
             qs0, qs1, gs0, gs1):
    c = lax.axis_index("c")
    s = lax.axis_index("s")
    lo = s * RPT

    pltpu.sync_copy(er_hbm.at[pl.ds(lo, RPT)], er_b.at[pl.ds(0, RPT)])
    pltpu.sync_copy(m_hbm, m_b)
    zv = jnp.zeros((L,), jnp.float32)
    er_b[pl.ds(RPT, L)] = zv
    m_sh = m_b[...]

    def zrow(r, _):
        for kk in range(D // L):
            acc[r, pl.ds(kk * L, L)] = zv
        return 0
    lax.fori_loop(0, ACCR, zrow, 0)

    iota = lax.iota(jnp.int32, L)
    colh = jnp.full((L,), H, jnp.int32)

    qsrc_ = (srcq0, srcq1)
    qdst_ = (dstq0, dstq1)
    qsem_ = (qs0, qs1)

    def stage_q(q, qb):
        pltpu.async_copy(src_hbm.at[c, q], qsrc_[qb], qsem_[qb])
        pltpu.async_copy(dst_hbm.at[c, q], qdst_[qb], qsem_[qb])

    def wait_q(qb):
        pltpu.make_async_copy(src_hbm.at[c, 0], qsrc_[qb], qsem_[qb]).wait()
        pltpu.make_async_copy(dst_hbm.at[c, 0], qdst_[qb], qsem_[qb]).wait()

    stage_q(0, 0)
    stage_q(1, 1)

    def scan_pair(q2, ptr):
        for qb in range(2):
            q = q2 * 2 + qb
            wait_q(qb)

            @plsc.parallel_loop(0, GQ // 2, step=1, carry=ptr)
            def group(g2x, p):
              for gu in range(2):
                g = g2x * 2 + gu
                srcv = qsrc_[qb][pl.ds(g * L, L)]
                dstv = qdst_[qb][pl.ds(g * L, L)]
                dl = dstv - lo
                msk = plsc.bitcast(dl, jnp.uint32) < jnp.uint32(RPT)
                packed = jnp.left_shift(srcv, 10) | dl
                plsc.store_compressed(list_b.at[pl.ds(p, L)], packed, mask=msk)
                p = p + plsc.all_reduce_population_count(msk)[0]
              return p

            ptr = group

            @pl.when(q + 2 < QN)
            def _():
                stage_q(q + 2, qb)
        return ptr

    ptr = lax.fori_loop(0, QN // 2, scan_pair, jnp.int32(0))

    sent = jnp.full((L,), RPT, jnp.int32)
    for w in range(2 * EC // L):
        list_b[pl.ds(ptr + w * L, L)] = sent
    nch2 = (ptr + 2 * EC - 1) // (2 * EC)

    sidx_ = (sidx0, sidx1)
    dstl_ = (dstl0, dstl1)
    ab_ = (ab0, ab1)
    row_ = (row0, row1)
    gsem_ = (gs0, gs1)

    def build(ch, b):
        for k in range(EC // L):
            v = list_b[pl.ds(ch * EC + k * L, L)]
            sidx_[b][pl.ds(k * L, L)] = jnp.right_shift(v, 10)
            dstl_[b][pl.ds(k * L, L)] = v & 1023
        pltpu.async_copy(feat_hbm.at[sidx_[b]], row_[b], gsem_[b])

    @pl.when(nch2 > 0)
    def _():
        build(0, 0)

    def chunk_pair(ch2, _):
        for b in range(2):
            ch = ch2 * 2 + b
            rb, dlb, ab = row_[b], dstl_[b], ab_[b]
            pltpu.make_async_copy(feat_hbm.at[sidx_[b]], rb, gsem_[b]).wait()

            @pl.when(ch + 1 < nch2 * 2)
            def _():
                build(ch + 1, 1 - b)

            for k in range(EC // L):
                elv = plsc.load_gather(rb, [iota + k * L, colh])
                erv = plsc.load_gather(er_b, [dlb[pl.ds(k * L, L)]])
                x = elv + erv
                e = jnp.where(x >= 0, x, 0.2 * x)
                a = jnp.exp(e - m_sh)
                ab[pl.ds(k * L, L)] = a

            @plsc.parallel_loop(0, EC // L, step=1)
            def egroup(k):
                a_v = ab[pl.ds(k * L, L)]
                dl_v = dlb[pl.ds(k * L, L)]
                for u in range(L):
                    t = k * L + u
                    a_s = a_v[u]
                    dl_s = dl_v[u]
                    for cc in range(H // L):
                        plsc.addupdate(acc.at[dl_s, pl.ds(cc * L, L)],
                                       rb[t, pl.ds(cc * L, L)] * a_s)
                    avu = jnp.where(iota == 0, a_s, 0.0)
                    plsc.addupdate(acc.at[dl_s, pl.ds(H, L)], avu)
        return 0

    lax.fori_loop(0, nch2, chunk_pair, 0)

    for z in range(WC):
        pltpu.sync_copy(acc.at[pl.ds(z * EC, EC)],
                        out_hbm.at[c, pl.ds(lo + z * EC, EC)])


_sc_gat = functools.partial(
    pl.kernel,
    out_type=jax.ShapeDtypeStruct((NC, NP, D), jnp.float32),
    mesh=plsc.VectorSubcoreMesh(core_axis_name="c", subcore_axis_name="s"),
    compiler_params=pltpu.CompilerParams(needs_layout_passes=False),
    scratch_types=[
        pltpu.VMEM((QC,), jnp.int32),
        pltpu.VMEM((QC,), jnp.int32),
        pltpu.VMEM((QC,), jnp.int32),
        pltpu.VMEM((QC,), jnp.int32),
        pltpu.VMEM((RPT + L,), jnp.float32),
        pltpu.VMEM((L,), jnp.float32),
        pltpu.VMEM((LSZ,), jnp.int32),
        pltpu.VMEM((EC,), jnp.int32),
        pltpu.VMEM((EC,), jnp.int32),
        pltpu.VMEM((EC + L,), jnp.int32),
        pltpu.VMEM((EC + L,), jnp.int32),
        pltpu.VMEM((EC + L,), jnp.float32),
        pltpu.VMEM((EC + L,), jnp.float32),
        pltpu.VMEM((EC, D), jnp.float32),
        pltpu.VMEM((EC, D), jnp.float32),
        pltpu.VMEM((ACCR, D), jnp.float32),
        pltpu.SemaphoreType.DMA,
        pltpu.SemaphoreType.DMA,
        pltpu.SemaphoreType.DMA,
        pltpu.SemaphoreType.DMA,
    ],
)(_sc_body)


def _head_body(s_ref, bias_ref, fcw_ref, fcb_ref, y_ref):
    num = s_ref[0, :N, :H] + s_ref[1, :N, :H]
    den = s_ref[0, :N, H:H + 1] + s_ref[1, :N, H:H + 1]
    rst = num / (den + 1e-16)
    h = rst + bias_ref[...]
    h = jnp.where(h > 0, h, jnp.exp(jnp.minimum(h, 0.0)) - 1.0)
    hg = jnp.mean(h, axis=0, keepdims=True)
    logit = jnp.sum(hg * fcw_ref[...], axis=1, keepdims=True) + fcb_ref[...]
    y_ref[...] = 1.0 / (1.0 + jnp.exp(-logit))


_head = pl.pallas_call(
    _head_body,
    out_shape=jax.ShapeDtypeStruct((1, 1), jnp.float32),
)


def kernel(features, edge_index, W, attn_l, attn_r, bias, fc_W, fc_b):
    al = attn_l.reshape(H)
    ar = attn_r.reshape(H)
    Wp = jnp.concatenate(
        [W, (W @ al)[:, None], jnp.zeros((F, D - H - 1), jnp.float32)], axis=1)
    alp = jnp.pad(al[:, None], ((0, D - H), (0, 0)))
    arp = jnp.pad(ar[:, None], ((0, D - H), (0, 0)))
    feat, er, m = _proj(features, Wp, alp, arp)
    er_p = jnp.pad(er.reshape(N), (0, NP - N))
    partials = _sc_gat(edge_index[0].reshape(NC, QN, QC),
                       edge_index[1].reshape(NC, QN, QC),
                       er_p, m.reshape(L), feat)
    return _head(partials, bias.reshape(1, H), fc_W, fc_b.reshape(1, 1))
